# baseline (speedup 1.0000x reference)
import functools

import jax
import jax.numpy as jnp
from jax import lax
from jax.experimental import pallas as pl
from jax.experimental.pallas import tpu as pltpu
from jax.experimental.pallas import tpu_sc as plsc

N = 10000
E = 320000
D = 128
NP = 10240
DUMMY = 10000
EP = 327680
EG = EP // 128
GPT = EG // 16
EG2 = EP // 64
GPT2 = EG2 // 16
ROWS_PT = NP // 16
DIAG_SCATTER = False

_MESH = plsc.VectorSubcoreMesh(core_axis_name="c", subcore_axis_name="s")


def _zero_buf(ref, rows):
    @pl.loop(0, rows)
    def _(r):
        for l in range(8):
            ref[r, pl.ds(l * 16, 16)] = jnp.zeros((16,), jnp.float32)


CBLK = 1024
NBLK_CAP = 21
CAP = NBLK_CAP * CBLK


@functools.partial(
    pl.kernel,
    out_type=(
        jax.ShapeDtypeStruct((2, 16, CAP), jnp.int32),
        jax.ShapeDtypeStruct((2, 16, CAP), jnp.int32),
        jax.ShapeDtypeStruct((2, 16, 16), jnp.int32),
        jax.ShapeDtypeStruct((2, NP), jnp.float32),
    ),
    mesh=_MESH,
    compiler_params=pltpu.CompilerParams(needs_layout_passes=False),
    scratch_types=[
        pltpu.VMEM((NP,), jnp.int32),
        pltpu.VMEM((16, 128), jnp.int32),
        pltpu.VMEM((16, 128), jnp.int32),
        pltpu.VMEM((16, 128), jnp.int32),
        pltpu.VMEM((2080,), jnp.int32),
        pltpu.VMEM((2080,), jnp.int32),
        pltpu.VMEM((128,), jnp.float32),
        pltpu.VMEM((ROWS_PT,), jnp.float32),
        pltpu.VMEM((16,), jnp.int32),
        pltpu.VMEM_SHARED((NP,), jnp.float32),
    ],
)
def _sc_stage1(src_h, dst_h, labels_h, csrc_h, cdst_h, cnt_h, degs_h,
               labels_v, sidx_v, dst_v, didx_v, st_s, st_d, ones_v, zer_v,
               cnt_v, deg_sp):
    c = lax.axis_index("c")
    s = lax.axis_index("s")

    @pl.loop(0, 8)
    def _(i):
        ones_v[pl.ds(i * 16, 16)] = jnp.full((16,), 1.0, jnp.float32)

    @pl.loop(0, ROWS_PT // 16)
    def _(i):
        zer_v[pl.ds(i * 16, 16)] = jnp.zeros((16,), jnp.float32)

    pltpu.sync_copy(labels_h, labels_v)
    pltpu.sync_copy(zer_v, deg_sp.at[pl.ds(s * ROWS_PT, ROWS_PT)])
    plsc.subcore_barrier()

    neg_core = c != 0
    gbase = s * GPT
    comp_s = csrc_h.at[c].at[s]
    comp_d = cdst_h.at[c].at[s]

    @pl.loop(0, GPT // 16, init_carry=(jnp.int32(0), jnp.int32(0)))
    def chunk_carry(it, carry):
        g0 = gbase + it * 16
        pltpu.sync_copy(src_h.at[pl.ds(g0, 16)], sidx_v)
        pltpu.sync_copy(dst_h.at[pl.ds(g0, 16)], dst_v)

        @pl.loop(0, 16, init_carry=carry)
        def group_carry(k, wb):
            wp, ob = wb
            for l in range(8):
                sl = sidx_v[k, pl.ds(l * 16, 16)]
                dl = dst_v[k, pl.ds(l * 16, 16)]
                ls = plsc.load_gather(labels_v, [sl])
                ld = plsc.load_gather(labels_v, [dl])
                want = (ls == ld) ^ neg_core
                didx_v[k, pl.ds(l * 16, 16)] = jnp.where(
                    want, dl, jnp.full((16,), DUMMY, jnp.int32))
                plsc.store_compressed(st_s.at[pl.ds(wp, 16)], sl, mask=want)
                plsc.store_compressed(st_d.at[pl.ds(wp, 16)], dl, mask=want)
                wp = wp + jnp.max(plsc.all_reduce_population_count(want))
            do_flush = wp >= CBLK

            @pl.when(do_flush)
            def _():
                pltpu.sync_copy(st_s.at[pl.ds(0, CBLK)],
                                comp_s.at[pl.ds(ob * CBLK, CBLK)])
                pltpu.sync_copy(st_d.at[pl.ds(0, CBLK)],
                                comp_d.at[pl.ds(ob * CBLK, CBLK)])
                nsh = (wp - CBLK + 15) // 16

                @pl.loop(0, nsh)
                def _(i):
                    st_s[pl.ds(i * 16, 16)] = st_s[pl.ds(CBLK + i * 16, 16)]
                    st_d[pl.ds(i * 16, 16)] = st_d[pl.ds(CBLK + i * 16, 16)]

            wp = jnp.where(do_flush, wp - CBLK, wp)
            ob = ob + do_flush.astype(jnp.int32)
            return (wp, ob)

        @pl.loop(0, 16)
        def _(k):
            pltpu.sync_copy(ones_v, deg_sp.at[didx_v.at[k]], add=True)

        return group_carry

    wp, ob = chunk_carry
    base16 = (wp // 16) * 16
    lane = lax.iota(jnp.int32, 16)
    keep = lane < (wp - base16)
    st_s[pl.ds(base16, 16)] = jnp.where(keep, st_s[pl.ds(base16, 16)], 0)
    st_d[pl.ds(base16, 16)] = jnp.where(keep, st_d[pl.ds(base16, 16)],
                                        jnp.full((16,), DUMMY, jnp.int32))

    @pl.loop(0, (CBLK - 16 - base16) // 16 + 1)
    def _(i):
        off = base16 + 16 + i * 16
        st_s[pl.ds(off, 16)] = jnp.zeros((16,), jnp.int32)
        st_d[pl.ds(off, 16)] = jnp.full((16,), DUMMY, jnp.int32)

    @pl.when(wp > 0)
    def _():
        pltpu.sync_copy(st_s.at[pl.ds(0, CBLK)],
                        comp_s.at[pl.ds(ob * CBLK, CBLK)])
        pltpu.sync_copy(st_d.at[pl.ds(0, CBLK)],
                        comp_d.at[pl.ds(ob * CBLK, CBLK)])

    nblk = ob + (wp > 0).astype(jnp.int32)
    cnt_v[pl.ds(0, 16)] = jnp.where(lane == 0, nblk, 0).astype(jnp.int32)
    pltpu.sync_copy(cnt_v, cnt_h.at[c].at[s])

    plsc.subcore_barrier()
    pltpu.sync_copy(deg_sp.at[pl.ds(s * ROWS_PT, ROWS_PT)],
                    degs_h.at[c].at[pl.ds(s * ROWS_PT, ROWS_PT)])


def _dexp_body(degs_ref, out_ref):
    d = degs_ref[...]
    dp = lax.rsqrt(jnp.clip(d[0:1], 1.0, None))
    dn = lax.rsqrt(jnp.clip(d[1:2], 1.0, None))
    da = lax.rsqrt(jnp.clip(d[0:1] + d[1:2], 1.0, None))
    stacked = jnp.concatenate([dp, dn, da], axis=0)
    out_ref[...] = jnp.broadcast_to(stacked[:, :, None], (3, NP, 128))


def _tc_dexp(degs):
    return pl.pallas_call(
        _dexp_body,
        out_shape=jax.ShapeDtypeStruct((3, NP, 128), jnp.float32),
    )(degs)


@functools.partial(
    pl.kernel,
    out_type=(
        jax.ShapeDtypeStruct((2, NP, 128), jnp.float32),
        jax.ShapeDtypeStruct((2, 4, NP, 128), jnp.float32),
        jax.ShapeDtypeStruct((2, NP, 128), jnp.float32),
    ),
    mesh=_MESH,
    compiler_params=pltpu.CompilerParams(needs_layout_passes=False),
    scratch_types=[
        pltpu.VMEM((64, 128), jnp.float32),
        pltpu.VMEM((64, 128), jnp.float32),
        pltpu.VMEM((64, 128), jnp.float32),
        pltpu.VMEM((64, 128), jnp.float32),
        pltpu.VMEM((16, 128), jnp.float32),
        pltpu.VMEM((16, 128), jnp.float32),
        pltpu.VMEM((16, 128), jnp.float32),
        pltpu.VMEM((16, 64), jnp.int32),
        pltpu.VMEM((16, 64), jnp.int32),
        pltpu.VMEM((16,), jnp.int32),
        pltpu.VMEM_SHARED((NP, 128), jnp.float32),
        [pltpu.SemaphoreType.DMA] * 4,
        [pltpu.SemaphoreType.DMA] * 4,
    ],
)
def _sc_stage3(feat_h, csrc_h, cdst_h, cnt_h, dexp_h, agg0_h, basis_h, t_h,
               rows0, rows1, rows2, rows3, u1, u2, u3, sidx_v, didx_v, cnt_v,
               acc_sp, gsems, ssems):
    c = lax.axis_index("c")
    s = lax.axis_index("s")
    rbase = s * ROWS_PT
    comp_s = csrc_h.at[c].at[s]
    comp_d = cdst_h.at[c].at[s]
    pltpu.sync_copy(cnt_h.at[c].at[s], cnt_v)
    nblk = cnt_v[pl.ds(0, 16)][0]
    nblk = jnp.maximum(nblk - 1, 0)

    @pl.loop(0, ROWS_PT // 16)
    def _(rc):
        b0 = rbase + rc * 16
        pltpu.sync_copy(feat_h.at[pl.ds(b0, 16)], u1)
        pltpu.sync_copy(dexp_h.at[c].at[pl.ds(b0, 16)], u3)

        @pl.loop(0, 16)
        def _(r):
            for l in range(8):
                sl = pl.ds(l * 16, 16)
                u2[r, sl] = u1[r, sl] * u3[r, sl]

        pltpu.sync_copy(u2, t_h.at[c].at[pl.ds(b0, 16)])

    def _zero_acc():
        _zero_buf(u1, 16)

        @pl.loop(0, ROWS_PT // 16)
        def _(rc):
            pltpu.sync_copy(u1, acc_sp.at[pl.ds(rbase + rc * 16, 16)])

    def _edge_pass(table_ref):
        bufs = (rows0, rows1, rows2, rows3)

        @pl.loop(0, nblk)
        def _(it):
            g0 = it * 16
            pltpu.sync_copy(comp_s.at[pl.ds(g0, 16)], sidx_v)
            pltpu.sync_copy(comp_d.at[pl.ds(g0, 16)], didx_v)
            gdesc = [None] * 4
            sdesc = [None] * 4
            for h in range(4):
                gdesc[h] = pltpu.async_copy(
                    table_ref.at[sidx_v.at[h]], bufs[h], gsems[h])
            for h in range(16):
                b = h & 3
                gdesc[b].wait()
                if DIAG_SCATTER:
                    sdesc[b] = pltpu.async_copy(
                        bufs[b], acc_sp.at[didx_v.at[h]], ssems[b], add=True)
                if h + 4 < 16:
                    if DIAG_SCATTER:
                        sdesc[b].wait()
                    gdesc[b] = pltpu.async_copy(
                        table_ref.at[sidx_v.at[h + 4]], bufs[b], gsems[b])
            if DIAG_SCATTER:
                for b in range(4):
                    sdesc[(12 + b) & 3].wait()

    _zero_acc()
    plsc.subcore_barrier()
    _edge_pass(feat_h)
    plsc.subcore_barrier()
    for rc in range(ROWS_PT // 128):
        b0 = rbase + rc * 128
        pltpu.sync_copy(acc_sp.at[pl.ds(b0, 128)],
                        agg0_h.at[c].at[pl.ds(b0, 128)])

    for j in range(1, 5):
        _zero_acc()
        plsc.subcore_barrier()
        _edge_pass(t_h.at[c])
        plsc.subcore_barrier()
        g_old = feat_h if j == 1 else basis_h.at[c].at[j - 2]
        basis_out = basis_h.at[c].at[j - 1]

        @pl.loop(0, ROWS_PT // 16)
        def _(rc):
            b0 = rbase + rc * 16
            pltpu.sync_copy(g_old.at[pl.ds(b0, 16)], u1)
            pltpu.sync_copy(acc_sp.at[pl.ds(b0, 16)], u2)
            pltpu.sync_copy(dexp_h.at[c].at[pl.ds(b0, 16)], u3)

            @pl.loop(0, 16)
            def _(r):
                for l in range(8):
                    sl = pl.ds(l * 16, 16)
                    gn = u1[r, sl] - u2[r, sl] * u3[r, sl]
                    u1[r, sl] = gn
                    u2[r, sl] = gn * u3[r, sl]

            pltpu.sync_copy(u1, basis_out.at[pl.ds(b0, 16)])
            pltpu.sync_copy(u2, t_h.at[c].at[pl.ds(b0, 16)])
        plsc.subcore_barrier()


_AF = (0.75, 0.75, 0.5, 0.6)
_AC = (0.0, -1.5, -2.75, -4.4)

_RB = 1000


def _final_body(feat_ref, agg0_ref, dexp_ref, basis_ref,
                wlin_ref, blin_ref, wlin1_ref, blin1_ref, wt_ref, bt_ref,
                hso_ref, hspn_ref, transh_ref):
    f = feat_ref[...]
    dall = dexp_ref[0]
    Cagg = (agg0_ref[0] + agg0_ref[1]) * dall

    wl = wlin_ref[...]
    wf = (_AF[0] * wl[0:128] + _AF[1] * wl[128:256]
          + _AF[2] * wl[256:384] + _AF[3] * wl[384:512])
    wc = (_AC[1] * wl[128:256] + _AC[2] * wl[256:384] + _AC[3] * wl[384:512])
    o = (jnp.dot(f, wf, preferred_element_type=jnp.float32)
         + jnp.dot(Cagg, wc, preferred_element_type=jnp.float32)
         + blin_ref[...])
    hso_ref[...] = jnp.where(o >= 0, o, 0.01 * o)

    g1, g2, g3, g4 = (basis_ref[0, i] for i in range(4))
    h1, h2, h3, h4 = (basis_ref[1, i] for i in range(4))
    hp0 = f - 0.5 * g1 + 0.25 * g2
    hp1 = 0.5 * g2 + 0.5 * g3 - 0.25 * g4
    hn0 = 0.25 * f - 0.25 * h1 + 0.5 * h2
    hn1 = 0.1 * h2 + 0.2 * h3 + 0.3 * h4
    w1 = wlin1_ref[...]
    pn = (jnp.dot(hp0, w1[0:128], preferred_element_type=jnp.float32)
          + jnp.dot(hp1, w1[128:256], preferred_element_type=jnp.float32)
          + jnp.dot(hn0, w1[256:384], preferred_element_type=jnp.float32)
          + jnp.dot(hn1, w1[384:512], preferred_element_type=jnp.float32)
          + blin1_ref[...])
    hspn_ref[...] = jnp.where(pn >= 0, pn, 0.01 * pn)

    t = (jnp.dot(f, wt_ref[...], preferred_element_type=jnp.float32)
         + bt_ref[...])
    transh_ref[...] = t


def _tc_final(feat, agg0c, dexp, basis, W_lin, b_lin, W_lin1, b_lin1,
              W_transh, b_transh):
    grid = N // _RB
    row_blk = lambda i: (i, 0)
    full2 = pl.BlockSpec((512, 128), lambda i: (0, 0))
    bias = pl.BlockSpec((1, 128), lambda i: (0, 0))
    return pl.pallas_call(
        _final_body,
        grid=(grid,),
        in_specs=[
            pl.BlockSpec((_RB, 128), row_blk),
            pl.BlockSpec((2, _RB, 128), lambda i: (0, i, 0)),
            pl.BlockSpec((1, _RB, 128), lambda i: (2, i, 0)),
            pl.BlockSpec((2, 4, _RB, 128), lambda i: (0, 0, i, 0)),
            full2, bias, full2, bias,
            pl.BlockSpec((128, 128), lambda i: (0, 0)), bias,
        ],
        out_specs=[
            pl.BlockSpec((_RB, 128), row_blk),
            pl.BlockSpec((_RB, 128), row_blk),
            pl.BlockSpec((_RB, 128), row_blk),
        ],
        out_shape=[
            jax.ShapeDtypeStruct((N, 128), jnp.float32),
            jax.ShapeDtypeStruct((N, 128), jnp.float32),
            jax.ShapeDtypeStruct((N, 128), jnp.float32),
        ],
    )(feat, agg0c, dexp, basis, W_lin, b_lin, W_lin1, b_lin1,
      W_transh, b_transh)


def kernel(feat, edge_index, labels, W_transh, b_transh, W_lin, b_lin,
           W_lin1, b_lin1):
    src = edge_index[0]
    dst = edge_index[1]
    src_p = jnp.pad(src, (0, EP - E)).reshape(EG, 128)
    dst_p = jnp.pad(dst, (0, EP - E), constant_values=DUMMY).reshape(EG, 128)
    labels_p = jnp.pad(labels, (0, NP - N))
    feat_p = jnp.pad(feat, ((0, NP - N), (0, 0)))

    csrc, cdst, cnts, degs = _sc_stage1(src_p, dst_p, labels_p)
    dexp = _tc_dexp(degs)
    agg0c, basis, _t = _sc_stage3(
        feat_p, csrc.reshape(2, 16, CAP // 64, 64),
        cdst.reshape(2, 16, CAP // 64, 64), cnts, dexp)
    hs_o, hs_pn, transh = _tc_final(
        feat, agg0c, dexp, basis,
        W_lin, b_lin.reshape(1, 128), W_lin1, b_lin1.reshape(1, 128),
        W_transh, b_transh.reshape(1, 128))
    return hs_o, hs_pn, transh


# speedup vs baseline: 2.0388x; 2.0388x over previous
import functools

import jax
import jax.numpy as jnp
from jax import lax
from jax.experimental import pallas as pl
from jax.experimental.pallas import tpu as pltpu
from jax.experimental.pallas import tpu_sc as plsc

N = 10000
E = 320000
D = 128
NP = 10240
DUMMY = 10000
EP = 327680
EG = EP // 128
GPT = EG // 16
EG2 = EP // 64
GPT2 = EG2 // 16
ROWS_PT = NP // 16

_MESH = plsc.VectorSubcoreMesh(core_axis_name="c", subcore_axis_name="s")


def _zero_buf(ref, rows):
    @pl.loop(0, rows)
    def _(r):
        for l in range(8):
            ref[r, pl.ds(l * 16, 16)] = jnp.zeros((16,), jnp.float32)


CBLK = 1024
NBLK_CAP = 21
CAP = NBLK_CAP * CBLK


@functools.partial(
    pl.kernel,
    out_type=(
        jax.ShapeDtypeStruct((2, 16, CAP), jnp.int32),
        jax.ShapeDtypeStruct((2, 16, CAP), jnp.int32),
        jax.ShapeDtypeStruct((2, 16, 16), jnp.int32),
        jax.ShapeDtypeStruct((2, NP), jnp.float32),
    ),
    mesh=_MESH,
    compiler_params=pltpu.CompilerParams(needs_layout_passes=False),
    scratch_types=[
        pltpu.VMEM((NP,), jnp.int32),
        pltpu.VMEM((16, 128), jnp.int32),
        pltpu.VMEM((16, 128), jnp.int32),
        pltpu.VMEM((16, 128), jnp.int32),
        pltpu.VMEM((2080,), jnp.int32),
        pltpu.VMEM((2080,), jnp.int32),
        pltpu.VMEM((128,), jnp.float32),
        pltpu.VMEM((ROWS_PT,), jnp.float32),
        pltpu.VMEM((16,), jnp.int32),
        pltpu.VMEM_SHARED((NP,), jnp.float32),
    ],
)
def _sc_stage1(src_h, dst_h, labels_h, csrc_h, cdst_h, cnt_h, degs_h,
               labels_v, sidx_v, dst_v, didx_v, st_s, st_d, ones_v, zer_v,
               cnt_v, deg_sp):
    c = lax.axis_index("c")
    s = lax.axis_index("s")

    @pl.loop(0, 8)
    def _(i):
        ones_v[pl.ds(i * 16, 16)] = jnp.full((16,), 1.0, jnp.float32)

    @pl.loop(0, ROWS_PT // 16)
    def _(i):
        zer_v[pl.ds(i * 16, 16)] = jnp.zeros((16,), jnp.float32)

    pltpu.sync_copy(labels_h, labels_v)
    pltpu.sync_copy(zer_v, deg_sp.at[pl.ds(s * ROWS_PT, ROWS_PT)])
    plsc.subcore_barrier()

    neg_core = c != 0
    gbase = s * GPT
    comp_s = csrc_h.at[c].at[s]
    comp_d = cdst_h.at[c].at[s]

    @pl.loop(0, GPT // 16, init_carry=(jnp.int32(0), jnp.int32(0)))
    def chunk_carry(it, carry):
        g0 = gbase + it * 16
        pltpu.sync_copy(src_h.at[pl.ds(g0, 16)], sidx_v)
        pltpu.sync_copy(dst_h.at[pl.ds(g0, 16)], dst_v)

        @pl.loop(0, 16, init_carry=carry)
        def group_carry(k, wb):
            wp, ob = wb
            for l in range(8):
                sl = sidx_v[k, pl.ds(l * 16, 16)]
                dl = dst_v[k, pl.ds(l * 16, 16)]
                ls = plsc.load_gather(labels_v, [sl])
                ld = plsc.load_gather(labels_v, [dl])
                want = ((ls == ld) ^ neg_core) & (dl < DUMMY)
                didx_v[k, pl.ds(l * 16, 16)] = jnp.where(
                    want, dl, jnp.full((16,), DUMMY, jnp.int32))
                plsc.store_compressed(st_s.at[pl.ds(wp, 16)], sl, mask=want)
                plsc.store_compressed(st_d.at[pl.ds(wp, 16)], dl, mask=want)
                wp = wp + jnp.max(plsc.all_reduce_population_count(want))
            do_flush = wp >= CBLK

            @pl.when(do_flush)
            def _():
                pltpu.sync_copy(st_s.at[pl.ds(0, CBLK)],
                                comp_s.at[pl.ds(ob * CBLK, CBLK)])
                pltpu.sync_copy(st_d.at[pl.ds(0, CBLK)],
                                comp_d.at[pl.ds(ob * CBLK, CBLK)])
                nsh = (wp - CBLK + 15) // 16

                @pl.loop(0, nsh)
                def _(i):
                    st_s[pl.ds(i * 16, 16)] = st_s[pl.ds(CBLK + i * 16, 16)]
                    st_d[pl.ds(i * 16, 16)] = st_d[pl.ds(CBLK + i * 16, 16)]

            wp = jnp.where(do_flush, wp - CBLK, wp)
            ob = ob + do_flush.astype(jnp.int32)
            return (wp, ob)

        @pl.loop(0, 16)
        def _(k):
            pltpu.sync_copy(ones_v, deg_sp.at[didx_v.at[k]], add=True)

        return group_carry

    wp, ob = chunk_carry
    base16 = (wp // 16) * 16
    lane = lax.iota(jnp.int32, 16)
    keep = lane < (wp - base16)
    dummy_s = lane * 64
    dummy_d = DUMMY + lane * 8
    st_s[pl.ds(base16, 16)] = jnp.where(keep, st_s[pl.ds(base16, 16)], dummy_s)
    st_d[pl.ds(base16, 16)] = jnp.where(keep, st_d[pl.ds(base16, 16)], dummy_d)

    @pl.loop(0, (CBLK - 16 - base16) // 16 + 1)
    def _(i):
        off = base16 + 16 + i * 16
        st_s[pl.ds(off, 16)] = lane * 64
        st_d[pl.ds(off, 16)] = DUMMY + lane * 8

    @pl.when(wp > 0)
    def _():
        pltpu.sync_copy(st_s.at[pl.ds(0, CBLK)],
                        comp_s.at[pl.ds(ob * CBLK, CBLK)])
        pltpu.sync_copy(st_d.at[pl.ds(0, CBLK)],
                        comp_d.at[pl.ds(ob * CBLK, CBLK)])

    nblk = ob + (wp > 0).astype(jnp.int32)
    cnt_v[pl.ds(0, 16)] = jnp.where(lane == 0, nblk, 0).astype(jnp.int32)
    pltpu.sync_copy(cnt_v, cnt_h.at[c].at[s])

    plsc.subcore_barrier()
    pltpu.sync_copy(deg_sp.at[pl.ds(s * ROWS_PT, ROWS_PT)],
                    degs_h.at[c].at[pl.ds(s * ROWS_PT, ROWS_PT)])


def _dexp_body(degs_ref, out_ref):
    d = degs_ref[...]
    dp = lax.rsqrt(jnp.clip(d[0:1], 1.0, None))
    dn = lax.rsqrt(jnp.clip(d[1:2], 1.0, None))
    da = lax.rsqrt(jnp.clip(d[0:1] + d[1:2], 1.0, None))
    stacked = jnp.concatenate([dp, dn, da], axis=0)
    out_ref[...] = jnp.broadcast_to(stacked[:, :, None], (3, NP, 128))


def _tc_dexp(degs):
    return pl.pallas_call(
        _dexp_body,
        out_shape=jax.ShapeDtypeStruct((3, NP, 128), jnp.float32),
    )(degs)


@functools.partial(
    pl.kernel,
    out_type=(
        jax.ShapeDtypeStruct((2, NP, 128), jnp.float32),
        jax.ShapeDtypeStruct((2, 4, NP, 128), jnp.float32),
        jax.ShapeDtypeStruct((2, NP, 128), jnp.float32),
    ),
    mesh=_MESH,
    compiler_params=pltpu.CompilerParams(needs_layout_passes=False),
    scratch_types=[
        pltpu.VMEM((64, 128), jnp.float32),
        pltpu.VMEM((64, 128), jnp.float32),
        pltpu.VMEM((64, 128), jnp.float32),
        pltpu.VMEM((64, 128), jnp.float32),
        pltpu.VMEM((16, 128), jnp.float32),
        pltpu.VMEM((16, 128), jnp.float32),
        pltpu.VMEM((16, 128), jnp.float32),
        pltpu.VMEM((16, 64), jnp.int32),
        pltpu.VMEM((16, 64), jnp.int32),
        pltpu.VMEM((16,), jnp.int32),
        pltpu.VMEM_SHARED((NP, 128), jnp.float32),
        [pltpu.SemaphoreType.DMA] * 4,
        [pltpu.SemaphoreType.DMA] * 4,
    ],
)
def _sc_stage3(feat_h, csrc_h, cdst_h, cnt_h, dexp_h, agg0_h, basis_h, t_h,
               rows0, rows1, rows2, rows3, u1, u2, u3, sidx_v, didx_v, cnt_v,
               acc_sp, gsems, ssems):
    c = lax.axis_index("c")
    s = lax.axis_index("s")
    rbase = s * ROWS_PT
    comp_s = csrc_h.at[c].at[s]
    comp_d = cdst_h.at[c].at[s]
    pltpu.sync_copy(cnt_h.at[c].at[s], cnt_v)
    nblk = cnt_v[pl.ds(0, 16)][0]

    @pl.loop(0, ROWS_PT // 16)
    def _(rc):
        b0 = rbase + rc * 16
        pltpu.sync_copy(feat_h.at[pl.ds(b0, 16)], u1)
        pltpu.sync_copy(dexp_h.at[c].at[pl.ds(b0, 16)], u3)

        @pl.loop(0, 16)
        def _(r):
            for l in range(8):
                sl = pl.ds(l * 16, 16)
                u2[r, sl] = u1[r, sl] * u3[r, sl]

        pltpu.sync_copy(u2, t_h.at[c].at[pl.ds(b0, 16)])

    def _zero_acc():
        _zero_buf(u1, 16)

        @pl.loop(0, ROWS_PT // 16)
        def _(rc):
            pltpu.sync_copy(u1, acc_sp.at[pl.ds(rbase + rc * 16, 16)])

    def _edge_pass(table_ref):
        bufs = (rows0, rows1, rows2, rows3)

        @pl.loop(0, nblk)
        def _(it):
            g0 = it * 16
            pltpu.sync_copy(comp_s.at[pl.ds(g0, 16)], sidx_v)
            pltpu.sync_copy(comp_d.at[pl.ds(g0, 16)], didx_v)
            gdesc = [None] * 4
            sdesc = [None] * 4
            for h in range(4):
                gdesc[h] = pltpu.async_copy(
                    table_ref.at[sidx_v.at[h]], bufs[h], gsems[h])
            for h in range(16):
                b = h & 3
                gdesc[b].wait()
                sdesc[b] = pltpu.async_copy(
                    bufs[b], acc_sp.at[didx_v.at[h]], ssems[b], add=True)
                if h + 4 < 16:
                    sdesc[b].wait()
                    gdesc[b] = pltpu.async_copy(
                        table_ref.at[sidx_v.at[h + 4]], bufs[b], gsems[b])
            for b in range(4):
                sdesc[(12 + b) & 3].wait()

    _zero_acc()
    plsc.subcore_barrier()
    _edge_pass(feat_h)
    plsc.subcore_barrier()
    for rc in range(ROWS_PT // 128):
        b0 = rbase + rc * 128
        pltpu.sync_copy(acc_sp.at[pl.ds(b0, 128)],
                        agg0_h.at[c].at[pl.ds(b0, 128)])

    for j in range(1, 5):
        _zero_acc()
        plsc.subcore_barrier()
        _edge_pass(t_h.at[c])
        plsc.subcore_barrier()
        g_old = feat_h if j == 1 else basis_h.at[c].at[j - 2]
        basis_out = basis_h.at[c].at[j - 1]

        @pl.loop(0, ROWS_PT // 16)
        def _(rc):
            b0 = rbase + rc * 16
            pltpu.sync_copy(g_old.at[pl.ds(b0, 16)], u1)
            pltpu.sync_copy(acc_sp.at[pl.ds(b0, 16)], u2)
            pltpu.sync_copy(dexp_h.at[c].at[pl.ds(b0, 16)], u3)

            @pl.loop(0, 16)
            def _(r):
                for l in range(8):
                    sl = pl.ds(l * 16, 16)
                    gn = u1[r, sl] - u2[r, sl] * u3[r, sl]
                    u1[r, sl] = gn
                    u2[r, sl] = gn * u3[r, sl]

            pltpu.sync_copy(u1, basis_out.at[pl.ds(b0, 16)])
            pltpu.sync_copy(u2, t_h.at[c].at[pl.ds(b0, 16)])
        plsc.subcore_barrier()


_AF = (0.75, 0.75, 0.5, 0.6)
_AC = (0.0, -1.5, -2.75, -4.4)

_RB = 1000


def _final_body(feat_ref, agg0_ref, dexp_ref, basis_ref,
                wlin_ref, blin_ref, wlin1_ref, blin1_ref, wt_ref, bt_ref,
                hso_ref, hspn_ref, transh_ref):
    f = feat_ref[...]
    dall = dexp_ref[0]
    Cagg = (agg0_ref[0] + agg0_ref[1]) * dall

    wl = wlin_ref[...]
    wf = (_AF[0] * wl[0:128] + _AF[1] * wl[128:256]
          + _AF[2] * wl[256:384] + _AF[3] * wl[384:512])
    wc = (_AC[1] * wl[128:256] + _AC[2] * wl[256:384] + _AC[3] * wl[384:512])
    o = (jnp.dot(f, wf, preferred_element_type=jnp.float32)
         + jnp.dot(Cagg, wc, preferred_element_type=jnp.float32)
         + blin_ref[...])
    hso_ref[...] = jnp.where(o >= 0, o, 0.01 * o)

    g1, g2, g3, g4 = (basis_ref[0, i] for i in range(4))
    h1, h2, h3, h4 = (basis_ref[1, i] for i in range(4))
    hp0 = f - 0.5 * g1 + 0.25 * g2
    hp1 = 0.5 * g2 + 0.5 * g3 - 0.25 * g4
    hn0 = 0.25 * f - 0.25 * h1 + 0.5 * h2
    hn1 = 0.1 * h2 + 0.2 * h3 + 0.3 * h4
    w1 = wlin1_ref[...]
    pn = (jnp.dot(hp0, w1[0:128], preferred_element_type=jnp.float32)
          + jnp.dot(hp1, w1[128:256], preferred_element_type=jnp.float32)
          + jnp.dot(hn0, w1[256:384], preferred_element_type=jnp.float32)
          + jnp.dot(hn1, w1[384:512], preferred_element_type=jnp.float32)
          + blin1_ref[...])
    hspn_ref[...] = jnp.where(pn >= 0, pn, 0.01 * pn)

    t = (jnp.dot(f, wt_ref[...], preferred_element_type=jnp.float32)
         + bt_ref[...])
    transh_ref[...] = t


def _tc_final(feat, agg0c, dexp, basis, W_lin, b_lin, W_lin1, b_lin1,
              W_transh, b_transh):
    grid = N // _RB
    row_blk = lambda i: (i, 0)
    full2 = pl.BlockSpec((512, 128), lambda i: (0, 0))
    bias = pl.BlockSpec((1, 128), lambda i: (0, 0))
    return pl.pallas_call(
        _final_body,
        grid=(grid,),
        in_specs=[
            pl.BlockSpec((_RB, 128), row_blk),
            pl.BlockSpec((2, _RB, 128), lambda i: (0, i, 0)),
            pl.BlockSpec((1, _RB, 128), lambda i: (2, i, 0)),
            pl.BlockSpec((2, 4, _RB, 128), lambda i: (0, 0, i, 0)),
            full2, bias, full2, bias,
            pl.BlockSpec((128, 128), lambda i: (0, 0)), bias,
        ],
        out_specs=[
            pl.BlockSpec((_RB, 128), row_blk),
            pl.BlockSpec((_RB, 128), row_blk),
            pl.BlockSpec((_RB, 128), row_blk),
        ],
        out_shape=[
            jax.ShapeDtypeStruct((N, 128), jnp.float32),
            jax.ShapeDtypeStruct((N, 128), jnp.float32),
            jax.ShapeDtypeStruct((N, 128), jnp.float32),
        ],
    )(feat, agg0c, dexp, basis, W_lin, b_lin, W_lin1, b_lin1,
      W_transh, b_transh)


def kernel(feat, edge_index, labels, W_transh, b_transh, W_lin, b_lin,
           W_lin1, b_lin1):
    src = edge_index[0]
    dst = edge_index[1]
    src_p = jnp.pad(src, (0, EP - E)).reshape(EG, 128)
    dst_p = jnp.pad(dst, (0, EP - E), constant_values=DUMMY).reshape(EG, 128)
    labels_p = jnp.pad(labels, (0, NP - N))
    feat_p = jnp.pad(feat, ((0, NP - N), (0, 0)))

    csrc, cdst, cnts, degs = _sc_stage1(src_p, dst_p, labels_p)
    dexp = _tc_dexp(degs)
    agg0c, basis, _t = _sc_stage3(
        feat_p, csrc.reshape(2, 16, CAP // 64, 64),
        cdst.reshape(2, 16, CAP // 64, 64), cnts, dexp)
    hs_o, hs_pn, transh = _tc_final(
        feat, agg0c, dexp, basis,
        W_lin, b_lin.reshape(1, 128), W_lin1, b_lin1.reshape(1, 128),
        W_transh, b_transh.reshape(1, 128))
    return hs_o, hs_pn, transh


# speedup vs baseline: 2.3907x; 1.1726x over previous
import functools

import jax
import jax.numpy as jnp
from jax import lax
from jax.experimental import pallas as pl
from jax.experimental.pallas import tpu as pltpu
from jax.experimental.pallas import tpu_sc as plsc

N = 10000
E = 320000
D = 128
NP = 10240
DUMMY = 10000
EP = 327680
EG = EP // 128
GPT = EG // 16
EG2 = EP // 64
GPT2 = EG2 // 16
ROWS_PT = NP // 16

_MESH = plsc.VectorSubcoreMesh(core_axis_name="c", subcore_axis_name="s")


def _zero_buf(ref, rows):
    @pl.loop(0, rows)
    def _(r):
        for l in range(8):
            ref[r, pl.ds(l * 16, 16)] = jnp.zeros((16,), jnp.float32)


CBLK = 1024
NBLK_CAP = 21
CAP = NBLK_CAP * CBLK


@functools.partial(
    pl.kernel,
    out_type=(
        jax.ShapeDtypeStruct((2, 16, CAP), jnp.int32),
        jax.ShapeDtypeStruct((2, 16, CAP), jnp.int32),
        jax.ShapeDtypeStruct((2, 16, 16), jnp.int32),
        jax.ShapeDtypeStruct((2, NP), jnp.float32),
    ),
    mesh=_MESH,
    compiler_params=pltpu.CompilerParams(needs_layout_passes=False),
    scratch_types=[
        pltpu.VMEM((NP,), jnp.int32),
        pltpu.VMEM((16, 128), jnp.int32),
        pltpu.VMEM((16, 128), jnp.int32),
        pltpu.VMEM((16, 128), jnp.int32),
        pltpu.VMEM((2080,), jnp.int32),
        pltpu.VMEM((2080,), jnp.int32),
        pltpu.VMEM((128,), jnp.float32),
        pltpu.VMEM((ROWS_PT,), jnp.float32),
        pltpu.VMEM((16,), jnp.int32),
        pltpu.VMEM_SHARED((NP,), jnp.float32),
    ],
)
def _sc_stage1(src_h, dst_h, labels_h, csrc_h, cdst_h, cnt_h, degs_h,
               labels_v, sidx_v, dst_v, didx_v, st_s, st_d, ones_v, zer_v,
               cnt_v, deg_sp):
    c = lax.axis_index("c")
    s = lax.axis_index("s")

    @pl.loop(0, 8)
    def _(i):
        ones_v[pl.ds(i * 16, 16)] = jnp.full((16,), 1.0, jnp.float32)

    @pl.loop(0, ROWS_PT // 16)
    def _(i):
        zer_v[pl.ds(i * 16, 16)] = jnp.zeros((16,), jnp.float32)

    pltpu.sync_copy(labels_h, labels_v)
    pltpu.sync_copy(zer_v, deg_sp.at[pl.ds(s * ROWS_PT, ROWS_PT)])
    plsc.subcore_barrier()

    neg_core = c != 0
    gbase = s * GPT
    comp_s = csrc_h.at[c].at[s]
    comp_d = cdst_h.at[c].at[s]

    @pl.loop(0, GPT // 16, init_carry=(jnp.int32(0), jnp.int32(0)))
    def chunk_carry(it, carry):
        g0 = gbase + it * 16
        pltpu.sync_copy(src_h.at[pl.ds(g0, 16)], sidx_v)
        pltpu.sync_copy(dst_h.at[pl.ds(g0, 16)], dst_v)

        @pl.loop(0, 16, init_carry=carry)
        def group_carry(k, wb):
            wp, ob = wb
            for l in range(8):
                sl = sidx_v[k, pl.ds(l * 16, 16)]
                dl = dst_v[k, pl.ds(l * 16, 16)]
                ls = plsc.load_gather(labels_v, [sl])
                ld = plsc.load_gather(labels_v, [dl])
                want = ((ls == ld) ^ neg_core) & (dl < DUMMY)
                didx_v[k, pl.ds(l * 16, 16)] = jnp.where(
                    want, dl, jnp.full((16,), DUMMY, jnp.int32))
                plsc.store_compressed(st_s.at[pl.ds(wp, 16)], sl, mask=want)
                plsc.store_compressed(st_d.at[pl.ds(wp, 16)], dl, mask=want)
                wp = wp + jnp.max(plsc.all_reduce_population_count(want))
            do_flush = wp >= CBLK

            @pl.when(do_flush)
            def _():
                pltpu.sync_copy(st_s.at[pl.ds(0, CBLK)],
                                comp_s.at[pl.ds(ob * CBLK, CBLK)])
                pltpu.sync_copy(st_d.at[pl.ds(0, CBLK)],
                                comp_d.at[pl.ds(ob * CBLK, CBLK)])
                nsh = (wp - CBLK + 15) // 16

                @pl.loop(0, nsh)
                def _(i):
                    st_s[pl.ds(i * 16, 16)] = st_s[pl.ds(CBLK + i * 16, 16)]
                    st_d[pl.ds(i * 16, 16)] = st_d[pl.ds(CBLK + i * 16, 16)]

            wp = jnp.where(do_flush, wp - CBLK, wp)
            ob = ob + do_flush.astype(jnp.int32)
            return (wp, ob)

        @pl.loop(0, 16)
        def _(k):
            pltpu.sync_copy(ones_v, deg_sp.at[didx_v.at[k]], add=True)

        return group_carry

    wp, ob = chunk_carry
    base16 = (wp // 16) * 16
    lane = lax.iota(jnp.int32, 16)
    keep = lane < (wp - base16)
    dummy_s = lane * 64
    dummy_d = DUMMY + lane * 8
    st_s[pl.ds(base16, 16)] = jnp.where(keep, st_s[pl.ds(base16, 16)], dummy_s)
    st_d[pl.ds(base16, 16)] = jnp.where(keep, st_d[pl.ds(base16, 16)], dummy_d)

    @pl.loop(0, (CBLK - 16 - base16) // 16 + 1)
    def _(i):
        off = base16 + 16 + i * 16
        st_s[pl.ds(off, 16)] = lane * 64
        st_d[pl.ds(off, 16)] = DUMMY + lane * 8

    @pl.when(wp > 0)
    def _():
        pltpu.sync_copy(st_s.at[pl.ds(0, CBLK)],
                        comp_s.at[pl.ds(ob * CBLK, CBLK)])
        pltpu.sync_copy(st_d.at[pl.ds(0, CBLK)],
                        comp_d.at[pl.ds(ob * CBLK, CBLK)])

    nblk = ob + (wp > 0).astype(jnp.int32)
    cnt_v[pl.ds(0, 16)] = jnp.where(lane == 0, nblk, 0).astype(jnp.int32)
    pltpu.sync_copy(cnt_v, cnt_h.at[c].at[s])

    plsc.subcore_barrier()
    pltpu.sync_copy(deg_sp.at[pl.ds(s * ROWS_PT, ROWS_PT)],
                    degs_h.at[c].at[pl.ds(s * ROWS_PT, ROWS_PT)])


def _dexp_body(degs_ref, out_ref):
    d = degs_ref[...]
    dp = lax.rsqrt(jnp.clip(d[0:1], 1.0, None))
    dn = lax.rsqrt(jnp.clip(d[1:2], 1.0, None))
    da = lax.rsqrt(jnp.clip(d[0:1] + d[1:2], 1.0, None))
    stacked = jnp.concatenate([dp, dn, da], axis=0)
    out_ref[...] = jnp.broadcast_to(stacked[:, :, None], (3, NP, 128))


def _tc_dexp(degs):
    return pl.pallas_call(
        _dexp_body,
        out_shape=jax.ShapeDtypeStruct((3, NP, 128), jnp.float32),
    )(degs)


@functools.partial(
    pl.kernel,
    out_type=(
        jax.ShapeDtypeStruct((2, NP, 128), jnp.float32),
        jax.ShapeDtypeStruct((2, 4, NP, 128), jnp.float32),
        jax.ShapeDtypeStruct((2, NP, 128), jnp.float32),
    ),
    mesh=_MESH,
    compiler_params=pltpu.CompilerParams(needs_layout_passes=False),
    scratch_types=[
        pltpu.VMEM((64, 128), jnp.float32),
        pltpu.VMEM((64, 128), jnp.float32),
        pltpu.VMEM((64, 128), jnp.float32),
        pltpu.VMEM((64, 128), jnp.float32),
        pltpu.VMEM((16, 64), jnp.int32),
        pltpu.VMEM((16, 64), jnp.int32),
        pltpu.VMEM((16,), jnp.int32),
        pltpu.VMEM_SHARED((NP, 128), jnp.float32),
        [pltpu.SemaphoreType.DMA] * 4,
        [pltpu.SemaphoreType.DMA] * 4,
    ],
)
def _sc_stage3(feat_h, csrc_h, cdst_h, cnt_h, dexp_h, agg0_h, basis_h, t_h,
               rows0, rows1, rows2, rows3, sidx_v, didx_v, cnt_v,
               acc_sp, gsems, ssems):
    c = lax.axis_index("c")
    s = lax.axis_index("s")
    rbase = s * ROWS_PT
    comp_s = csrc_h.at[c].at[s]
    comp_d = cdst_h.at[c].at[s]
    pltpu.sync_copy(cnt_h.at[c].at[s], cnt_v)
    nblk = cnt_v[pl.ds(0, 16)][0]

    @pl.loop(0, ROWS_PT // 64)
    def _(rc):
        b0 = rbase + rc * 64
        pltpu.sync_copy(feat_h.at[pl.ds(b0, 64)], rows0)
        pltpu.sync_copy(dexp_h.at[c].at[pl.ds(b0, 64)], rows2)

        @pl.loop(0, 64)
        def _(r):
            for l in range(8):
                sl = pl.ds(l * 16, 16)
                rows1[r, sl] = rows0[r, sl] * rows2[r, sl]

        pltpu.sync_copy(rows1, t_h.at[c].at[pl.ds(b0, 64)])

    def _zero_acc():
        _zero_buf(rows0, 64)

        @pl.loop(0, ROWS_PT // 64)
        def _(rc):
            pltpu.sync_copy(rows0, acc_sp.at[pl.ds(rbase + rc * 64, 64)])

    def _edge_pass(table_ref):
        bufs = (rows0, rows1, rows2, rows3)

        @pl.loop(0, nblk)
        def _(it):
            g0 = it * 16
            pltpu.sync_copy(comp_s.at[pl.ds(g0, 16)], sidx_v)
            pltpu.sync_copy(comp_d.at[pl.ds(g0, 16)], didx_v)
            gdesc = [None] * 4
            sdesc = [None] * 4
            for h in range(4):
                gdesc[h] = pltpu.async_copy(
                    table_ref.at[sidx_v.at[h]], bufs[h], gsems[h])
            for h in range(16):
                b = h & 3
                gdesc[b].wait()
                sdesc[b] = pltpu.async_copy(
                    bufs[b], acc_sp.at[didx_v.at[h]], ssems[b], add=True)
                if h + 4 < 16:
                    sdesc[b].wait()
                    gdesc[b] = pltpu.async_copy(
                        table_ref.at[sidx_v.at[h + 4]], bufs[b], gsems[b])
            for b in range(4):
                sdesc[(12 + b) & 3].wait()

    _zero_acc()
    plsc.subcore_barrier()
    _edge_pass(feat_h)
    plsc.subcore_barrier()
    for rc in range(ROWS_PT // 128):
        b0 = rbase + rc * 128
        pltpu.sync_copy(acc_sp.at[pl.ds(b0, 128)],
                        agg0_h.at[c].at[pl.ds(b0, 128)])

    for j in range(1, 5):
        _zero_acc()
        plsc.subcore_barrier()
        _edge_pass(t_h.at[c])
        plsc.subcore_barrier()
        g_old = feat_h if j == 1 else basis_h.at[c].at[j - 2]
        basis_out = basis_h.at[c].at[j - 1]

        @pl.loop(0, ROWS_PT // 64)
        def _(rc):
            b0 = rbase + rc * 64
            pltpu.sync_copy(g_old.at[pl.ds(b0, 64)], rows0)
            pltpu.sync_copy(acc_sp.at[pl.ds(b0, 64)], rows1)
            pltpu.sync_copy(dexp_h.at[c].at[pl.ds(b0, 64)], rows2)

            @pl.loop(0, 64)
            def _(r):
                for l in range(8):
                    sl = pl.ds(l * 16, 16)
                    gn = rows0[r, sl] - rows1[r, sl] * rows2[r, sl]
                    rows3[r, sl] = gn
                    rows0[r, sl] = gn * rows2[r, sl]

            pltpu.sync_copy(rows3, basis_out.at[pl.ds(b0, 64)])
            pltpu.sync_copy(rows0, t_h.at[c].at[pl.ds(b0, 64)])
        plsc.subcore_barrier()


_AF = (0.75, 0.75, 0.5, 0.6)
_AC = (0.0, -1.5, -2.75, -4.4)

_RB = 1000


def _final_body(feat_ref, agg0_ref, dexp_ref, basis_ref,
                wlin_ref, blin_ref, wlin1_ref, blin1_ref, wt_ref, bt_ref,
                hso_ref, hspn_ref, transh_ref):
    f = feat_ref[...]
    dall = dexp_ref[0]
    Cagg = (agg0_ref[0] + agg0_ref[1]) * dall

    wl = wlin_ref[...]
    wf = (_AF[0] * wl[0:128] + _AF[1] * wl[128:256]
          + _AF[2] * wl[256:384] + _AF[3] * wl[384:512])
    wc = (_AC[1] * wl[128:256] + _AC[2] * wl[256:384] + _AC[3] * wl[384:512])
    o = (jnp.dot(f, wf, preferred_element_type=jnp.float32)
         + jnp.dot(Cagg, wc, preferred_element_type=jnp.float32)
         + blin_ref[...])
    hso_ref[...] = jnp.where(o >= 0, o, 0.01 * o)

    g1, g2, g3, g4 = (basis_ref[0, i] for i in range(4))
    h1, h2, h3, h4 = (basis_ref[1, i] for i in range(4))
    hp0 = f - 0.5 * g1 + 0.25 * g2
    hp1 = 0.5 * g2 + 0.5 * g3 - 0.25 * g4
    hn0 = 0.25 * f - 0.25 * h1 + 0.5 * h2
    hn1 = 0.1 * h2 + 0.2 * h3 + 0.3 * h4
    w1 = wlin1_ref[...]
    pn = (jnp.dot(hp0, w1[0:128], preferred_element_type=jnp.float32)
          + jnp.dot(hp1, w1[128:256], preferred_element_type=jnp.float32)
          + jnp.dot(hn0, w1[256:384], preferred_element_type=jnp.float32)
          + jnp.dot(hn1, w1[384:512], preferred_element_type=jnp.float32)
          + blin1_ref[...])
    hspn_ref[...] = jnp.where(pn >= 0, pn, 0.01 * pn)

    t = (jnp.dot(f, wt_ref[...], preferred_element_type=jnp.float32)
         + bt_ref[...])
    transh_ref[...] = t


def _tc_final(feat, agg0c, dexp, basis, W_lin, b_lin, W_lin1, b_lin1,
              W_transh, b_transh):
    grid = N // _RB
    row_blk = lambda i: (i, 0)
    full2 = pl.BlockSpec((512, 128), lambda i: (0, 0))
    bias = pl.BlockSpec((1, 128), lambda i: (0, 0))
    return pl.pallas_call(
        _final_body,
        grid=(grid,),
        in_specs=[
            pl.BlockSpec((_RB, 128), row_blk),
            pl.BlockSpec((2, _RB, 128), lambda i: (0, i, 0)),
            pl.BlockSpec((1, _RB, 128), lambda i: (2, i, 0)),
            pl.BlockSpec((2, 4, _RB, 128), lambda i: (0, 0, i, 0)),
            full2, bias, full2, bias,
            pl.BlockSpec((128, 128), lambda i: (0, 0)), bias,
        ],
        out_specs=[
            pl.BlockSpec((_RB, 128), row_blk),
            pl.BlockSpec((_RB, 128), row_blk),
            pl.BlockSpec((_RB, 128), row_blk),
        ],
        out_shape=[
            jax.ShapeDtypeStruct((N, 128), jnp.float32),
            jax.ShapeDtypeStruct((N, 128), jnp.float32),
            jax.ShapeDtypeStruct((N, 128), jnp.float32),
        ],
    )(feat, agg0c, dexp, basis, W_lin, b_lin, W_lin1, b_lin1,
      W_transh, b_transh)


def kernel(feat, edge_index, labels, W_transh, b_transh, W_lin, b_lin,
           W_lin1, b_lin1):
    src = edge_index[0]
    dst = edge_index[1]
    src_p = jnp.pad(src, (0, EP - E)).reshape(EG, 128)
    dst_p = jnp.pad(dst, (0, EP - E), constant_values=DUMMY).reshape(EG, 128)
    labels_p = jnp.pad(labels, (0, NP - N))
    feat_p = jnp.pad(feat, ((0, NP - N), (0, 0)))

    csrc, cdst, cnts, degs = _sc_stage1(src_p, dst_p, labels_p)
    dexp = _tc_dexp(degs)
    agg0c, basis, _t = _sc_stage3(
        feat_p, csrc.reshape(2, 16, CAP // 64, 64),
        cdst.reshape(2, 16, CAP // 64, 64), cnts, dexp)
    hs_o, hs_pn, transh = _tc_final(
        feat, agg0c, dexp, basis,
        W_lin, b_lin.reshape(1, 128), W_lin1, b_lin1.reshape(1, 128),
        W_transh, b_transh.reshape(1, 128))
    return hs_o, hs_pn, transh


# speedup vs baseline: 2.3910x; 1.0001x over previous
import functools

import jax
import jax.numpy as jnp
from jax import lax
from jax.experimental import pallas as pl
from jax.experimental.pallas import tpu as pltpu
from jax.experimental.pallas import tpu_sc as plsc

N = 10000
E = 320000
D = 128
NP = 10240
DUMMY = 10000
EP = 327680
EG = EP // 128
GPT = EG // 16
EG2 = EP // 64
GPT2 = EG2 // 16
ROWS_PT = NP // 16

_MESH = plsc.VectorSubcoreMesh(core_axis_name="c", subcore_axis_name="s")


def _zero_buf(ref, rows):
    @pl.loop(0, rows)
    def _(r):
        for l in range(8):
            ref[r, pl.ds(l * 16, 16)] = jnp.zeros((16,), jnp.float32)


CBLK = 1024
NBLK_CAP = 21
CAP = NBLK_CAP * CBLK


@functools.partial(
    pl.kernel,
    out_type=(
        jax.ShapeDtypeStruct((2, 16, CAP), jnp.int32),
        jax.ShapeDtypeStruct((2, 16, CAP), jnp.int32),
        jax.ShapeDtypeStruct((2, 16, 16), jnp.int32),
        jax.ShapeDtypeStruct((2, NP), jnp.float32),
    ),
    mesh=_MESH,
    compiler_params=pltpu.CompilerParams(needs_layout_passes=False),
    scratch_types=[
        pltpu.VMEM((NP,), jnp.int32),
        pltpu.VMEM((16, 128), jnp.int32),
        pltpu.VMEM((16, 128), jnp.int32),
        pltpu.VMEM((16, 128), jnp.int32),
        pltpu.VMEM((2080,), jnp.int32),
        pltpu.VMEM((2080,), jnp.int32),
        pltpu.VMEM((128,), jnp.float32),
        pltpu.VMEM((ROWS_PT,), jnp.float32),
        pltpu.VMEM((16,), jnp.int32),
        pltpu.VMEM_SHARED((NP,), jnp.float32),
        pltpu.SemaphoreType.DMA,
    ],
)
def _sc_stage1(src_h, dst_h, labels_h, csrc_h, cdst_h, cnt_h, degs_h,
               labels_v, sidx_v, dst_v, didx_v, st_s, st_d, ones_v, zer_v,
               cnt_v, deg_sp, dsem):
    c = lax.axis_index("c")
    s = lax.axis_index("s")

    @pl.loop(0, 8)
    def _(i):
        ones_v[pl.ds(i * 16, 16)] = jnp.full((16,), 1.0, jnp.float32)

    @pl.loop(0, ROWS_PT // 16)
    def _(i):
        zer_v[pl.ds(i * 16, 16)] = jnp.zeros((16,), jnp.float32)

    pltpu.sync_copy(labels_h, labels_v)
    pltpu.sync_copy(zer_v, deg_sp.at[pl.ds(s * ROWS_PT, ROWS_PT)])
    plsc.subcore_barrier()

    neg_core = c != 0
    gbase = s * GPT
    comp_s = csrc_h.at[c].at[s]
    comp_d = cdst_h.at[c].at[s]

    @pl.loop(0, GPT // 16, init_carry=(jnp.int32(0), jnp.int32(0)))
    def chunk_carry(it, carry):
        g0 = gbase + it * 16
        pltpu.sync_copy(src_h.at[pl.ds(g0, 16)], sidx_v)
        pltpu.sync_copy(dst_h.at[pl.ds(g0, 16)], dst_v)

        @pl.loop(0, 16, init_carry=carry)
        def group_carry(k, wb):
            wp, ob = wb
            for l in range(8):
                sl = sidx_v[k, pl.ds(l * 16, 16)]
                dl = dst_v[k, pl.ds(l * 16, 16)]
                ls = plsc.load_gather(labels_v, [sl])
                ld = plsc.load_gather(labels_v, [dl])
                want = ((ls == ld) ^ neg_core) & (dl < DUMMY)
                didx_v[k, pl.ds(l * 16, 16)] = jnp.where(
                    want, dl, jnp.full((16,), DUMMY, jnp.int32))
                plsc.store_compressed(st_s.at[pl.ds(wp, 16)], sl, mask=want)
                plsc.store_compressed(st_d.at[pl.ds(wp, 16)], dl, mask=want)
                wp = wp + jnp.max(plsc.all_reduce_population_count(want))
            do_flush = wp >= CBLK

            @pl.when(do_flush)
            def _():
                pltpu.sync_copy(st_s.at[pl.ds(0, CBLK)],
                                comp_s.at[pl.ds(ob * CBLK, CBLK)])
                pltpu.sync_copy(st_d.at[pl.ds(0, CBLK)],
                                comp_d.at[pl.ds(ob * CBLK, CBLK)])
                nsh = (wp - CBLK + 15) // 16

                @pl.loop(0, nsh)
                def _(i):
                    st_s[pl.ds(i * 16, 16)] = st_s[pl.ds(CBLK + i * 16, 16)]
                    st_d[pl.ds(i * 16, 16)] = st_d[pl.ds(CBLK + i * 16, 16)]

            wp = jnp.where(do_flush, wp - CBLK, wp)
            ob = ob + do_flush.astype(jnp.int32)
            return (wp, ob)

        descs = [pltpu.async_copy(ones_v, deg_sp.at[didx_v.at[k]], dsem,
                                  add=True) for k in range(16)]
        for dd in descs:
            dd.wait()

        return group_carry

    wp, ob = chunk_carry
    base16 = (wp // 16) * 16
    lane = lax.iota(jnp.int32, 16)
    keep = lane < (wp - base16)
    dummy_s = lane * 64
    dummy_d = DUMMY + lane * 8
    st_s[pl.ds(base16, 16)] = jnp.where(keep, st_s[pl.ds(base16, 16)], dummy_s)
    st_d[pl.ds(base16, 16)] = jnp.where(keep, st_d[pl.ds(base16, 16)], dummy_d)

    @pl.loop(0, (CBLK - 16 - base16) // 16 + 1)
    def _(i):
        off = base16 + 16 + i * 16
        st_s[pl.ds(off, 16)] = lane * 64
        st_d[pl.ds(off, 16)] = DUMMY + lane * 8

    @pl.when(wp > 0)
    def _():
        pltpu.sync_copy(st_s.at[pl.ds(0, CBLK)],
                        comp_s.at[pl.ds(ob * CBLK, CBLK)])
        pltpu.sync_copy(st_d.at[pl.ds(0, CBLK)],
                        comp_d.at[pl.ds(ob * CBLK, CBLK)])

    nblk = ob + (wp > 0).astype(jnp.int32)
    cnt_v[pl.ds(0, 16)] = jnp.where(lane == 0, nblk, 0).astype(jnp.int32)
    pltpu.sync_copy(cnt_v, cnt_h.at[c].at[s])

    plsc.subcore_barrier()
    pltpu.sync_copy(deg_sp.at[pl.ds(s * ROWS_PT, ROWS_PT)],
                    degs_h.at[c].at[pl.ds(s * ROWS_PT, ROWS_PT)])


def _dexp_body(degs_ref, out_ref):
    d = degs_ref[...]
    dp = lax.rsqrt(jnp.clip(d[0:1], 1.0, None))
    dn = lax.rsqrt(jnp.clip(d[1:2], 1.0, None))
    da = lax.rsqrt(jnp.clip(d[0:1] + d[1:2], 1.0, None))
    stacked = jnp.concatenate([dp, dn, da], axis=0)
    out_ref[...] = jnp.broadcast_to(stacked[:, :, None], (3, NP, 128))


def _tc_dexp(degs):
    return pl.pallas_call(
        _dexp_body,
        out_shape=jax.ShapeDtypeStruct((3, NP, 128), jnp.float32),
    )(degs)


@functools.partial(
    pl.kernel,
    out_type=(
        jax.ShapeDtypeStruct((2, NP, 128), jnp.float32),
        jax.ShapeDtypeStruct((2, 4, NP, 128), jnp.float32),
        jax.ShapeDtypeStruct((2, NP, 128), jnp.float32),
    ),
    mesh=_MESH,
    compiler_params=pltpu.CompilerParams(needs_layout_passes=False),
    scratch_types=[
        pltpu.VMEM((64, 128), jnp.float32),
        pltpu.VMEM((64, 128), jnp.float32),
        pltpu.VMEM((64, 128), jnp.float32),
        pltpu.VMEM((64, 128), jnp.float32),
        pltpu.VMEM((16, 64), jnp.int32),
        pltpu.VMEM((16, 64), jnp.int32),
        pltpu.VMEM((16,), jnp.int32),
        pltpu.VMEM_SHARED((NP, 128), jnp.float32),
        [pltpu.SemaphoreType.DMA] * 4,
        [pltpu.SemaphoreType.DMA] * 4,
    ],
)
def _sc_stage3(feat_h, csrc_h, cdst_h, cnt_h, dexp_h, agg0_h, basis_h, t_h,
               rows0, rows1, rows2, rows3, sidx_v, didx_v, cnt_v,
               acc_sp, gsems, ssems):
    c = lax.axis_index("c")
    s = lax.axis_index("s")
    rbase = s * ROWS_PT
    comp_s = csrc_h.at[c].at[s]
    comp_d = cdst_h.at[c].at[s]
    pltpu.sync_copy(cnt_h.at[c].at[s], cnt_v)
    nblk = cnt_v[pl.ds(0, 16)][0]

    @pl.loop(0, ROWS_PT // 64)
    def _(rc):
        b0 = rbase + rc * 64
        pltpu.sync_copy(feat_h.at[pl.ds(b0, 64)], rows0)
        pltpu.sync_copy(dexp_h.at[c].at[pl.ds(b0, 64)], rows2)

        @pl.loop(0, 64)
        def _(r):
            for l in range(8):
                sl = pl.ds(l * 16, 16)
                rows1[r, sl] = rows0[r, sl] * rows2[r, sl]

        pltpu.sync_copy(rows1, t_h.at[c].at[pl.ds(b0, 64)])

    def _zero_acc():
        _zero_buf(rows0, 64)

        @pl.loop(0, ROWS_PT // 64)
        def _(rc):
            pltpu.sync_copy(rows0, acc_sp.at[pl.ds(rbase + rc * 64, 64)])

    def _edge_pass(table_ref):
        bufs = (rows0, rows1, rows2, rows3)

        @pl.loop(0, nblk)
        def _(it):
            g0 = it * 16
            pltpu.sync_copy(comp_s.at[pl.ds(g0, 16)], sidx_v)
            pltpu.sync_copy(comp_d.at[pl.ds(g0, 16)], didx_v)
            gdesc = [None] * 4
            sdesc = [None] * 4
            for h in range(4):
                gdesc[h] = pltpu.async_copy(
                    table_ref.at[sidx_v.at[h]], bufs[h], gsems[h])
            for h in range(16):
                b = h & 3
                gdesc[b].wait()
                sdesc[b] = pltpu.async_copy(
                    bufs[b], acc_sp.at[didx_v.at[h]], ssems[b], add=True)
                if h + 4 < 16:
                    sdesc[b].wait()
                    gdesc[b] = pltpu.async_copy(
                        table_ref.at[sidx_v.at[h + 4]], bufs[b], gsems[b])
            for b in range(4):
                sdesc[(12 + b) & 3].wait()

    _zero_acc()
    plsc.subcore_barrier()
    _edge_pass(feat_h)
    plsc.subcore_barrier()
    for rc in range(ROWS_PT // 128):
        b0 = rbase + rc * 128
        pltpu.sync_copy(acc_sp.at[pl.ds(b0, 128)],
                        agg0_h.at[c].at[pl.ds(b0, 128)])

    for j in range(1, 5):
        _zero_acc()
        plsc.subcore_barrier()
        _edge_pass(t_h.at[c])
        plsc.subcore_barrier()
        g_old = feat_h if j == 1 else basis_h.at[c].at[j - 2]
        basis_out = basis_h.at[c].at[j - 1]

        @pl.loop(0, ROWS_PT // 64)
        def _(rc):
            b0 = rbase + rc * 64
            pltpu.sync_copy(g_old.at[pl.ds(b0, 64)], rows0)
            pltpu.sync_copy(acc_sp.at[pl.ds(b0, 64)], rows1)
            pltpu.sync_copy(dexp_h.at[c].at[pl.ds(b0, 64)], rows2)

            @pl.loop(0, 64)
            def _(r):
                for l in range(8):
                    sl = pl.ds(l * 16, 16)
                    gn = rows0[r, sl] - rows1[r, sl] * rows2[r, sl]
                    rows3[r, sl] = gn
                    rows0[r, sl] = gn * rows2[r, sl]

            pltpu.sync_copy(rows3, basis_out.at[pl.ds(b0, 64)])
            pltpu.sync_copy(rows0, t_h.at[c].at[pl.ds(b0, 64)])
        plsc.subcore_barrier()


_AF = (0.75, 0.75, 0.5, 0.6)
_AC = (0.0, -1.5, -2.75, -4.4)

_RB = 1000


def _final_body(feat_ref, agg0_ref, dexp_ref, basis_ref,
                wlin_ref, blin_ref, wlin1_ref, blin1_ref, wt_ref, bt_ref,
                hso_ref, hspn_ref, transh_ref):
    f = feat_ref[...]
    dall = dexp_ref[0]
    Cagg = (agg0_ref[0] + agg0_ref[1]) * dall

    wl = wlin_ref[...]
    wf = (_AF[0] * wl[0:128] + _AF[1] * wl[128:256]
          + _AF[2] * wl[256:384] + _AF[3] * wl[384:512])
    wc = (_AC[1] * wl[128:256] + _AC[2] * wl[256:384] + _AC[3] * wl[384:512])
    o = (jnp.dot(f, wf, preferred_element_type=jnp.float32)
         + jnp.dot(Cagg, wc, preferred_element_type=jnp.float32)
         + blin_ref[...])
    hso_ref[...] = jnp.where(o >= 0, o, 0.01 * o)

    g1, g2, g3, g4 = (basis_ref[0, i] for i in range(4))
    h1, h2, h3, h4 = (basis_ref[1, i] for i in range(4))
    hp0 = f - 0.5 * g1 + 0.25 * g2
    hp1 = 0.5 * g2 + 0.5 * g3 - 0.25 * g4
    hn0 = 0.25 * f - 0.25 * h1 + 0.5 * h2
    hn1 = 0.1 * h2 + 0.2 * h3 + 0.3 * h4
    w1 = wlin1_ref[...]
    pn = (jnp.dot(hp0, w1[0:128], preferred_element_type=jnp.float32)
          + jnp.dot(hp1, w1[128:256], preferred_element_type=jnp.float32)
          + jnp.dot(hn0, w1[256:384], preferred_element_type=jnp.float32)
          + jnp.dot(hn1, w1[384:512], preferred_element_type=jnp.float32)
          + blin1_ref[...])
    hspn_ref[...] = jnp.where(pn >= 0, pn, 0.01 * pn)

    t = (jnp.dot(f, wt_ref[...], preferred_element_type=jnp.float32)
         + bt_ref[...])
    transh_ref[...] = t


def _tc_final(feat, agg0c, dexp, basis, W_lin, b_lin, W_lin1, b_lin1,
              W_transh, b_transh):
    grid = N // _RB
    row_blk = lambda i: (i, 0)
    full2 = pl.BlockSpec((512, 128), lambda i: (0, 0))
    bias = pl.BlockSpec((1, 128), lambda i: (0, 0))
    return pl.pallas_call(
        _final_body,
        grid=(grid,),
        in_specs=[
            pl.BlockSpec((_RB, 128), row_blk),
            pl.BlockSpec((2, _RB, 128), lambda i: (0, i, 0)),
            pl.BlockSpec((1, _RB, 128), lambda i: (2, i, 0)),
            pl.BlockSpec((2, 4, _RB, 128), lambda i: (0, 0, i, 0)),
            full2, bias, full2, bias,
            pl.BlockSpec((128, 128), lambda i: (0, 0)), bias,
        ],
        out_specs=[
            pl.BlockSpec((_RB, 128), row_blk),
            pl.BlockSpec((_RB, 128), row_blk),
            pl.BlockSpec((_RB, 128), row_blk),
        ],
        out_shape=[
            jax.ShapeDtypeStruct((N, 128), jnp.float32),
            jax.ShapeDtypeStruct((N, 128), jnp.float32),
            jax.ShapeDtypeStruct((N, 128), jnp.float32),
        ],
    )(feat, agg0c, dexp, basis, W_lin, b_lin, W_lin1, b_lin1,
      W_transh, b_transh)


def kernel(feat, edge_index, labels, W_transh, b_transh, W_lin, b_lin,
           W_lin1, b_lin1):
    src = edge_index[0]
    dst = edge_index[1]
    src_p = jnp.pad(src, (0, EP - E)).reshape(EG, 128)
    dst_p = jnp.pad(dst, (0, EP - E), constant_values=DUMMY).reshape(EG, 128)
    labels_p = jnp.pad(labels, (0, NP - N))
    feat_p = jnp.pad(feat, ((0, NP - N), (0, 0)))

    csrc, cdst, cnts, degs = _sc_stage1(src_p, dst_p, labels_p)
    dexp = _tc_dexp(degs)
    agg0c, basis, _t = _sc_stage3(
        feat_p, csrc.reshape(2, 16, CAP // 64, 64),
        cdst.reshape(2, 16, CAP // 64, 64), cnts, dexp)
    hs_o, hs_pn, transh = _tc_final(
        feat, agg0c, dexp, basis,
        W_lin, b_lin.reshape(1, 128), W_lin1, b_lin1.reshape(1, 128),
        W_transh, b_transh.reshape(1, 128))
    return hs_o, hs_pn, transh
